# baseline (speedup 1.0000x reference)
import numpy as np
import jax
import jax.numpy as jnp
from jax.experimental import pallas as pl
from jax.experimental.pallas import tpu as pltpu

_B = 128
_N = 100000
_SUB = 8
_NSTRIP = _B // _SUB
_TILE = 640
_NFULL = _N // _TILE
_OVER = ((_N - _TILE + 127) // 128) * 128
_NPAD = _OVER + _TILE

_TINY = np.float32(np.finfo(np.float32).tiny)
_NEG_INF = np.float32(-np.inf)
_INT_MAX = np.int32(2**31 - 1)

_KS1 = np.uint32(42)
_KS2 = np.uint32(42 ^ 0x1BD11BDA)


def _threefry_bits(x1):
    def rotl(x, r):
        return (x << np.uint32(r)) | (x >> np.uint32(32 - r))

    x0 = x1
    x1 = rotl(x1, 13) ^ x0
    for r in (15, 26, 6):
        x0 = x0 + x1
        x1 = rotl(x1, r) ^ x0
    x0 = x0 + _KS1
    x1 = x1 + np.uint32(_KS2 + np.uint32(1))

    for r in (17, 29, 16, 24):
        x0 = x0 + x1
        x1 = rotl(x1, r) ^ x0
    x0 = x0 + _KS2
    x1 = x1 + np.uint32(2)

    for r in (13, 15, 26, 6):
        x0 = x0 + x1
        x1 = rotl(x1, r) ^ x0
    x1 = x1 + np.uint32(_KS1 + np.uint32(3))

    for r in (17, 29, 16, 24):
        x0 = x0 + x1
        x1 = rotl(x1, r) ^ x0
    x0 = x0 + _KS1
    x1 = x1 + np.uint32(_KS2 + np.uint32(4))

    for r in (13, 15, 26, 6):
        x0 = x0 + x1
        x1 = rotl(x1, r) ^ x0
    x0 = x0 + _KS2
    x1 = x1 + np.uint32(5)

    return x0 ^ x1


def _sample_kernel(logits_ref, out_ref):
    s = pl.program_id(0)

    lane = jax.lax.broadcasted_iota(jnp.int32, (_SUB, _TILE), 1)
    rowoff = (jax.lax.broadcasted_iota(jnp.int32, (_SUB, _TILE), 0) * _N
              + (s * (_SUB * _N) + 42))
    laneoff = lane + rowoff

    def tile_val(off):
        x1i = laneoff + off
        bits = _threefry_bits(x1i.astype(jnp.uint32))
        fbits = (bits >> np.uint32(9)) | np.uint32(0x3F800000)
        m01 = (jax.lax.bitcast_convert_type(fbits, jnp.float32)
               - np.float32(1.0))
        u = jnp.maximum(m01, _TINY)
        t4 = jnp.log(-jnp.log(u))
        return logits_ref[:, pl.ds(off, _TILE)] - t4, x1i

    def tile_body(t, carry):
        acc_val, acc_idx = carry
        val, x1i = tile_val(pl.multiple_of(t * _TILE, 128))
        upd = val > acc_val
        acc_val = jnp.maximum(acc_val, val)
        acc_idx = jnp.where(upd, x1i, acc_idx)
        return acc_val, acc_idx

    acc_val = jnp.full((_SUB, _TILE), _NEG_INF, jnp.float32)
    acc_idx = jnp.zeros((_SUB, _TILE), jnp.int32)
    acc_val, acc_idx = jax.lax.fori_loop(
        0, _NFULL, tile_body, (acc_val, acc_idx), unroll=False)

    val, x1i = tile_val(_OVER)
    val = jnp.where(lane + _OVER < _N, val, _NEG_INF)
    upd = val > acc_val
    acc_val = jnp.maximum(acc_val, val)
    acc_idx = jnp.where(upd, x1i, acc_idx)

    m = jnp.max(acc_val, axis=1, keepdims=True)
    cand = jnp.where(acc_val == m, acc_idx, _INT_MAX)
    x1min = jnp.min(cand, axis=1, keepdims=True)
    rowbase = (jax.lax.broadcasted_iota(jnp.int32, (_SUB, 1), 0) * _N
               + (s * (_SUB * _N) + 42))
    out_ref[...] = x1min - rowbase


def kernel(logits):
    out = pl.pallas_call(
        _sample_kernel,
        grid=(_NSTRIP,),
        in_specs=[pl.BlockSpec((_SUB, _NPAD), lambda s: (s, 0))],
        out_specs=pl.BlockSpec((_SUB, 1), lambda s: (s, 0)),
        out_shape=jax.ShapeDtypeStruct((_B, 1), jnp.int32),
        compiler_params=pltpu.CompilerParams(
            dimension_semantics=("arbitrary",),
        ),
    )(logits)
    return out.reshape(_B)


# speedup vs baseline: 1.2393x; 1.2393x over previous
import numpy as np
import jax
import jax.numpy as jnp
from jax.experimental import pallas as pl
from jax.experimental.pallas import tpu as pltpu

_B = 128
_N = 100000
_SUB = 8
_NSTRIP = _B // _SUB
_TILE = 1280
_NFULL = _N // _TILE
_OVER = ((_N - _TILE + 127) // 128) * 128
_NPAD = _OVER + _TILE

_TINY = np.float32(np.finfo(np.float32).tiny)
_NEG_INF = np.float32(-np.inf)
_INT_MAX = np.int32(2**31 - 1)

_KS1 = np.uint32(42)
_KS2 = np.uint32(42 ^ 0x1BD11BDA)


def _threefry_bits(x1):
    def rotl(x, r):
        return (x << np.uint32(r)) | (x >> np.uint32(32 - r))

    x0 = x1
    x1 = rotl(x1, 13) ^ x0
    for r in (15, 26, 6):
        x0 = x0 + x1
        x1 = rotl(x1, r) ^ x0
    x0 = x0 + _KS1
    x1 = x1 + np.uint32(_KS2 + np.uint32(1))

    for r in (17, 29, 16, 24):
        x0 = x0 + x1
        x1 = rotl(x1, r) ^ x0
    x0 = x0 + _KS2
    x1 = x1 + np.uint32(2)

    for r in (13, 15, 26, 6):
        x0 = x0 + x1
        x1 = rotl(x1, r) ^ x0
    x1 = x1 + np.uint32(_KS1 + np.uint32(3))

    for r in (17, 29, 16, 24):
        x0 = x0 + x1
        x1 = rotl(x1, r) ^ x0
    x0 = x0 + _KS1
    x1 = x1 + np.uint32(_KS2 + np.uint32(4))

    for r in (13, 15, 26, 6):
        x0 = x0 + x1
        x1 = rotl(x1, r) ^ x0
    x0 = x0 + _KS2
    x1 = x1 + np.uint32(5)

    return x0 ^ x1


def _sample_kernel(logits_ref, out_ref):
    s = pl.program_id(0)

    lane = jax.lax.broadcasted_iota(jnp.int32, (_SUB, _TILE), 1)
    rowoff = (jax.lax.broadcasted_iota(jnp.int32, (_SUB, _TILE), 0) * _N
              + (s * (_SUB * _N) + 42))
    laneoff = lane + rowoff

    def tile_val(off):
        x1i = laneoff + off
        bits = _threefry_bits(x1i.astype(jnp.uint32))
        fbits = (bits >> np.uint32(9)) | np.uint32(0x3F800000)
        m01 = (jax.lax.bitcast_convert_type(fbits, jnp.float32)
               - np.float32(1.0))
        u = jnp.maximum(m01, _TINY)
        t4 = jnp.log(-jnp.log(u))
        return logits_ref[:, pl.ds(off, _TILE)] - t4, x1i

    def tile_body(t, carry):
        acc_val, acc_idx = carry
        val, x1i = tile_val(pl.multiple_of(t * _TILE, 128))
        upd = val > acc_val
        acc_val = jnp.maximum(acc_val, val)
        acc_idx = jnp.where(upd, x1i, acc_idx)
        return acc_val, acc_idx

    acc_val = jnp.full((_SUB, _TILE), _NEG_INF, jnp.float32)
    acc_idx = jnp.zeros((_SUB, _TILE), jnp.int32)
    acc_val, acc_idx = jax.lax.fori_loop(
        0, _NFULL, tile_body, (acc_val, acc_idx), unroll=False)

    val, x1i = tile_val(_OVER)
    val = jnp.where(lane + _OVER < _N, val, _NEG_INF)
    upd = val > acc_val
    acc_val = jnp.maximum(acc_val, val)
    acc_idx = jnp.where(upd, x1i, acc_idx)

    m = jnp.max(acc_val, axis=1, keepdims=True)
    cand = jnp.where(acc_val == m, acc_idx, _INT_MAX)
    x1min = jnp.min(cand, axis=1, keepdims=True)
    rowbase = (jax.lax.broadcasted_iota(jnp.int32, (_SUB, 1), 0) * _N
               + (s * (_SUB * _N) + 42))
    out_ref[...] = x1min - rowbase


def kernel(logits):
    out = pl.pallas_call(
        _sample_kernel,
        grid=(_NSTRIP,),
        in_specs=[pl.BlockSpec((_SUB, _NPAD), lambda s: (s, 0))],
        out_specs=pl.BlockSpec((_SUB, 1), lambda s: (s, 0)),
        out_shape=jax.ShapeDtypeStruct((_B, 1), jnp.int32),
        compiler_params=pltpu.CompilerParams(
            dimension_semantics=("arbitrary",),
        ),
    )(logits)
    return out.reshape(_B)


# speedup vs baseline: 1.2928x; 1.0432x over previous
import numpy as np
import jax
import jax.numpy as jnp
from jax.experimental import pallas as pl
from jax.experimental.pallas import tpu as pltpu

_B = 128
_N = 100000
_CHUNK = 25600
_GRID = (_N + _CHUNK - 1) // _CHUNK
_SUB = 8
_TILE = 1280
_NSTRIP = _B // _SUB
_NTILE = _CHUNK // _TILE

_LAST_FULL = (_N - (_GRID - 1) * _CHUNK) // _TILE

_TINY = np.float32(np.finfo(np.float32).tiny)
_NEG_INF = np.float32(-np.inf)
_INT_MAX = np.int32(2**31 - 1)

_KS1 = np.uint32(42)
_KS2 = np.uint32(42 ^ 0x1BD11BDA)


def _threefry_bits(x1):
    def rotl(x, r):
        return (x << np.uint32(r)) | (x >> np.uint32(32 - r))

    x0 = x1
    x1 = rotl(x1, 13) ^ x0
    for r in (15, 26, 6):
        x0 = x0 + x1
        x1 = rotl(x1, r) ^ x0
    x0 = x0 + _KS1
    x1 = x1 + np.uint32(_KS2 + np.uint32(1))

    for r in (17, 29, 16, 24):
        x0 = x0 + x1
        x1 = rotl(x1, r) ^ x0
    x0 = x0 + _KS2
    x1 = x1 + np.uint32(2)

    for r in (13, 15, 26, 6):
        x0 = x0 + x1
        x1 = rotl(x1, r) ^ x0
    x1 = x1 + np.uint32(_KS1 + np.uint32(3))

    for r in (17, 29, 16, 24):
        x0 = x0 + x1
        x1 = rotl(x1, r) ^ x0
    x0 = x0 + _KS1
    x1 = x1 + np.uint32(_KS2 + np.uint32(4))

    for r in (13, 15, 26, 6):
        x0 = x0 + x1
        x1 = rotl(x1, r) ^ x0
    x0 = x0 + _KS2
    x1 = x1 + np.uint32(5)

    return x0 ^ x1


def _sample_kernel(logits_ref, out_ref, acc_val_ref, acc_idx_ref):
    j = pl.program_id(0)
    chunk_base = j * _CHUNK
    is_last = j == _GRID - 1

    lane = jax.lax.broadcasted_iota(jnp.int32, (_SUB, _TILE), 1)
    row_iota = jax.lax.broadcasted_iota(jnp.int32, (_SUB, _TILE), 0) * _N

    @pl.when(j == 0)
    def _init():
        acc_val_ref[...] = jnp.full((_B, _TILE), _NEG_INF, jnp.float32)
        acc_idx_ref[...] = jnp.zeros((_B, _TILE), jnp.int32)

    def strip_body(s, _):
        row0 = s * _SUB
        rows = pl.ds(row0, _SUB)
        laneoff = lane + row_iota + (row0 * _N + 42)

        acc_val = acc_val_ref[rows, :]
        acc_idx = acc_idx_ref[rows, :]

        for t in range(_NTILE):
            off = t * _TILE
            x1i = laneoff + (chunk_base + off)
            bits = _threefry_bits(x1i.astype(jnp.uint32))

            fbits = (bits >> np.uint32(9)) | np.uint32(0x3F800000)
            m01 = (jax.lax.bitcast_convert_type(fbits, jnp.float32)
                   - np.float32(1.0))
            u = jnp.maximum(m01, _TINY)
            t4 = jnp.log(-jnp.log(u))

            val = logits_ref[rows, pl.ds(off, _TILE)] - t4
            if t >= _LAST_FULL:
                val = jnp.where(lane + (chunk_base + off) < _N,
                                val, _NEG_INF)

            upd = val > acc_val
            acc_val = jnp.maximum(acc_val, val)
            acc_idx = jnp.where(upd, x1i, acc_idx)

        acc_val_ref[rows, :] = acc_val
        acc_idx_ref[rows, :] = acc_idx

        @pl.when(is_last)
        def _finish():
            m = jnp.max(acc_val, axis=1, keepdims=True)
            cand = jnp.where(acc_val == m, acc_idx, _INT_MAX)
            x1min = jnp.min(cand, axis=1, keepdims=True)
            rowbase = (jax.lax.broadcasted_iota(jnp.int32, (_SUB, 1), 0) * _N
                       + (row0 * _N + 42))
            out_ref[rows, :] = x1min - rowbase

        return 0

    jax.lax.fori_loop(0, _NSTRIP, strip_body, 0, unroll=False)


def kernel(logits):
    out = pl.pallas_call(
        _sample_kernel,
        grid=(_GRID,),
        in_specs=[pl.BlockSpec((_B, _CHUNK), lambda j: (0, j))],
        out_specs=pl.BlockSpec((_B, 1), lambda j: (0, 0)),
        out_shape=jax.ShapeDtypeStruct((_B, 1), jnp.int32),
        scratch_shapes=[
            pltpu.VMEM((_B, _TILE), jnp.float32),
            pltpu.VMEM((_B, _TILE), jnp.int32),
        ],
        compiler_params=pltpu.CompilerParams(
            dimension_semantics=("arbitrary",),
        ),
    )(logits)
    return out.reshape(_B)


# speedup vs baseline: 1.2935x; 1.0006x over previous
import numpy as np
import jax
import jax.numpy as jnp
from jax.experimental import pallas as pl
from jax.experimental.pallas import tpu as pltpu

_B = 128
_N = 100000
_CHUNK = 25600
_GRID = (_N + _CHUNK - 1) // _CHUNK
_SUB = 8
_TILE = 1280
_NSTRIP = _B // _SUB
_NTILE = _CHUNK // _TILE

_LAST_FULL = (_N - (_GRID - 1) * _CHUNK) // _TILE

_TINY = np.float32(np.finfo(np.float32).tiny)
_NEG_INF = np.float32(-np.inf)
_INT_MAX = np.int32(2**31 - 1)

_KS1 = np.uint32(42)
_KS2 = np.uint32(42 ^ 0x1BD11BDA)


def _threefry_bits(x1):
    def rotl(x, r):
        return (x << np.uint32(r)) | (x >> np.uint32(32 - r))

    x0 = x1
    x1 = rotl(x1, 13) ^ x0
    for r in (15, 26, 6):
        x0 = x0 + x1
        x1 = rotl(x1, r) ^ x0
    x0 = x0 + _KS1
    x1 = x1 + np.uint32(_KS2 + np.uint32(1))

    for r in (17, 29, 16, 24):
        x0 = x0 + x1
        x1 = rotl(x1, r) ^ x0
    x0 = x0 + _KS2
    x1 = x1 + np.uint32(2)

    for r in (13, 15, 26, 6):
        x0 = x0 + x1
        x1 = rotl(x1, r) ^ x0
    x1 = x1 + np.uint32(_KS1 + np.uint32(3))

    for r in (17, 29, 16, 24):
        x0 = x0 + x1
        x1 = rotl(x1, r) ^ x0
    x0 = x0 + _KS1
    x1 = x1 + np.uint32(_KS2 + np.uint32(4))

    for r in (13, 15, 26, 6):
        x0 = x0 + x1
        x1 = rotl(x1, r) ^ x0
    x0 = x0 + _KS2
    x1 = x1 + np.uint32(5)

    return x0 ^ x1


def _sample_kernel(logits_ref, out_ref, acc_val_ref, acc_idx_ref):
    j = pl.program_id(0)
    chunk_base = j * _CHUNK
    is_last = j == _GRID - 1

    lane = jax.lax.broadcasted_iota(jnp.int32, (_SUB, _TILE), 1)
    row_iota = jax.lax.broadcasted_iota(jnp.int32, (_SUB, _TILE), 0) * _N

    @pl.when(j == 0)
    def _init():
        acc_val_ref[...] = jnp.full((_B, _TILE), _NEG_INF, jnp.float32)
        acc_idx_ref[...] = jnp.zeros((_B, _TILE), jnp.int32)

    def strip_body(s, _):
        row0 = s * _SUB
        rows = pl.ds(row0, _SUB)
        laneoff = lane + row_iota + (row0 * _N + 42)

        acc_val = acc_val_ref[rows, :]
        acc_idx = acc_idx_ref[rows, :]

        for t in range(_NTILE):
            off = t * _TILE
            x1i = laneoff + (chunk_base + off)
            bits = _threefry_bits(x1i.astype(jnp.uint32))

            fbits = (bits >> np.uint32(9)) | np.uint32(0x3F800000)
            m01 = (jax.lax.bitcast_convert_type(fbits, jnp.float32)
                   - np.float32(1.0))
            u = jnp.maximum(m01, _TINY)
            t4 = jnp.log(-jnp.log(u))

            val = logits_ref[rows, pl.ds(off, _TILE)] - t4
            if t >= _LAST_FULL:
                val = jnp.where(lane + (chunk_base + off) < _N,
                                val, _NEG_INF)

            upd = val > acc_val
            acc_val = jnp.maximum(acc_val, val)
            acc_idx = jnp.where(upd, x1i, acc_idx)

        acc_val_ref[rows, :] = acc_val
        acc_idx_ref[rows, :] = acc_idx

        @pl.when(is_last)
        def _finish():
            m = jnp.max(acc_val, axis=1, keepdims=True)
            cand = jnp.where(acc_val == m, acc_idx, _INT_MAX)
            x1min = jnp.min(cand, axis=1, keepdims=True)
            rowbase = (jax.lax.broadcasted_iota(jnp.int32, (_SUB, 1), 0) * _N
                       + (row0 * _N + 42))
            out_ref[rows, :] = x1min - rowbase

        return 0

    jax.lax.fori_loop(0, _NSTRIP, strip_body, 0, unroll=2)


def kernel(logits):
    out = pl.pallas_call(
        _sample_kernel,
        grid=(_GRID,),
        in_specs=[pl.BlockSpec((_B, _CHUNK), lambda j: (0, j))],
        out_specs=pl.BlockSpec((_B, 1), lambda j: (0, 0)),
        out_shape=jax.ShapeDtypeStruct((_B, 1), jnp.int32),
        scratch_shapes=[
            pltpu.VMEM((_B, _TILE), jnp.float32),
            pltpu.VMEM((_B, _TILE), jnp.int32),
        ],
        compiler_params=pltpu.CompilerParams(
            dimension_semantics=("arbitrary",),
        ),
    )(logits)
    return out.reshape(_B)


# speedup vs baseline: 1.2944x; 1.0007x over previous
import numpy as np
import jax
import jax.numpy as jnp
from jax.experimental import pallas as pl
from jax.experimental.pallas import tpu as pltpu

_B = 128
_N = 100000
_CHUNK = 25600
_GRID = (_N + _CHUNK - 1) // _CHUNK
_SUB = 8
_TILE = 1280
_NSTRIP = _B // _SUB
_NTILE = _CHUNK // _TILE

_LAST_FULL = (_N - (_GRID - 1) * _CHUNK) // _TILE

_TINY = np.float32(np.finfo(np.float32).tiny)
_NEG_INF = np.float32(-np.inf)
_INT_MAX = np.int32(2**31 - 1)

_KS1 = np.uint32(42)
_KS2 = np.uint32(42 ^ 0x1BD11BDA)


def _threefry_bits(x1):
    def rotl(x, r):
        return x * np.uint32(1 << r) + (x >> np.uint32(32 - r))

    x0 = x1
    x1 = rotl(x1, 13) ^ x0
    for r in (15, 26, 6):
        x0 = x0 + x1
        x1 = rotl(x1, r) ^ x0
    x0 = x0 + _KS1
    x1 = x1 + np.uint32(_KS2 + np.uint32(1))

    for r in (17, 29, 16, 24):
        x0 = x0 + x1
        x1 = rotl(x1, r) ^ x0
    x0 = x0 + _KS2
    x1 = x1 + np.uint32(2)

    for r in (13, 15, 26, 6):
        x0 = x0 + x1
        x1 = rotl(x1, r) ^ x0
    x1 = x1 + np.uint32(_KS1 + np.uint32(3))

    for r in (17, 29, 16, 24):
        x0 = x0 + x1
        x1 = rotl(x1, r) ^ x0
    x0 = x0 + _KS1
    x1 = x1 + np.uint32(_KS2 + np.uint32(4))

    for r in (13, 15, 26, 6):
        x0 = x0 + x1
        x1 = rotl(x1, r) ^ x0
    x0 = x0 + _KS2
    x1 = x1 + np.uint32(5)

    return x0 ^ x1


def _sample_kernel(logits_ref, out_ref, acc_val_ref, acc_idx_ref):
    j = pl.program_id(0)
    chunk_base = j * _CHUNK
    is_last = j == _GRID - 1

    lane = jax.lax.broadcasted_iota(jnp.int32, (_SUB, _TILE), 1)
    row_iota = jax.lax.broadcasted_iota(jnp.int32, (_SUB, _TILE), 0) * _N

    @pl.when(j == 0)
    def _init():
        acc_val_ref[...] = jnp.full((_B, _TILE), _NEG_INF, jnp.float32)
        acc_idx_ref[...] = jnp.zeros((_B, _TILE), jnp.int32)

    def strip_body(s, _):
        row0 = s * _SUB
        rows = pl.ds(row0, _SUB)
        laneoff = lane + row_iota + (row0 * _N + 42)

        acc_val = acc_val_ref[rows, :]
        acc_idx = acc_idx_ref[rows, :]

        for t in range(_NTILE):
            off = t * _TILE
            x1i = laneoff + (chunk_base + off)
            bits = _threefry_bits(x1i.astype(jnp.uint32))

            fbits = (bits >> np.uint32(9)) | np.uint32(0x3F800000)
            m01 = (jax.lax.bitcast_convert_type(fbits, jnp.float32)
                   - np.float32(1.0))
            u = jnp.maximum(m01, _TINY)
            t4 = jnp.log(-jnp.log(u))

            val = logits_ref[rows, pl.ds(off, _TILE)] - t4
            if t >= _LAST_FULL:
                val = jnp.where(lane + (chunk_base + off) < _N,
                                val, _NEG_INF)

            upd = val > acc_val
            acc_val = jnp.maximum(acc_val, val)
            acc_idx = jnp.where(upd, x1i, acc_idx)

        acc_val_ref[rows, :] = acc_val
        acc_idx_ref[rows, :] = acc_idx

        @pl.when(is_last)
        def _finish():
            m = jnp.max(acc_val, axis=1, keepdims=True)
            cand = jnp.where(acc_val == m, acc_idx, _INT_MAX)
            x1min = jnp.min(cand, axis=1, keepdims=True)
            rowbase = (jax.lax.broadcasted_iota(jnp.int32, (_SUB, 1), 0) * _N
                       + (row0 * _N + 42))
            out_ref[rows, :] = x1min - rowbase

        return 0

    jax.lax.fori_loop(0, _NSTRIP, strip_body, 0, unroll=2)


def kernel(logits):
    out = pl.pallas_call(
        _sample_kernel,
        grid=(_GRID,),
        in_specs=[pl.BlockSpec((_B, _CHUNK), lambda j: (0, j))],
        out_specs=pl.BlockSpec((_B, 1), lambda j: (0, 0)),
        out_shape=jax.ShapeDtypeStruct((_B, 1), jnp.int32),
        scratch_shapes=[
            pltpu.VMEM((_B, _TILE), jnp.float32),
            pltpu.VMEM((_B, _TILE), jnp.int32),
        ],
        compiler_params=pltpu.CompilerParams(
            dimension_semantics=("arbitrary",),
        ),
    )(logits)
    return out.reshape(_B)
